# Initial kernel scaffold; baseline (speedup 1.0000x reference)
import jax
import jax.numpy as jnp
from jax.experimental import pallas as pl


def kernel(x, edge_index, W1, b1, W2, b2, Wa, ba):
    raise NotImplementedError("write your pallas kernel here")


# speedup vs baseline: 23.7479x; 23.7479x over previous
import functools

import jax
import jax.numpy as jnp
from jax import lax
from jax.experimental import pallas as pl
from jax.experimental.pallas import tpu as pltpu
from jax.experimental.pallas import tpu_sc as plsc

N = 10000
E = 320000
NC = 2
NS = 16
NW = NC * NS
CHUNK = 80
CH = E // NW // CHUNK
NP = 10240
RPS = NP // NS
BLK = 1000


def _vmesh():
    return plsc.VectorSubcoreMesh(core_axis_name="c", subcore_axis_name="s")


_SC_PARAMS = pltpu.CompilerParams(use_tc_tiling_on_sc=False)


def _deg_pass(zeros16, dst3):
    @functools.partial(
        pl.kernel,
        out_type=jax.ShapeDtypeStruct((NC, NP, 16), jnp.float32),
        mesh=_vmesh(),
        scratch_types=[
            pltpu.VMEM((CH, CHUNK), jnp.int32),
            pltpu.VMEM((CHUNK, 16), jnp.float32),
            pltpu.VMEM_SHARED((NP, 16), jnp.float32),
        ],
        compiler_params=_SC_PARAMS,
    )
    def k(zeros_hbm, dst_hbm, out_hbm, dstv, ones_v, acc_sp):
        c = lax.axis_index("c")
        s = lax.axis_index("s")
        w = c * NS + s
        row = jnp.where(lax.iota(jnp.int32, 16) == 0, 1.0, 0.0).astype(jnp.float32)

        @pl.loop(0, CHUNK)
        def _(i):
            ones_v[i] = row

        pltpu.sync_copy(zeros_hbm.at[pl.ds(s * RPS, RPS)],
                        acc_sp.at[pl.ds(s * RPS, RPS)])
        pltpu.sync_copy(dst_hbm.at[w], dstv)
        plsc.subcore_barrier()

        @pl.loop(0, CH)
        def _(j):
            pltpu.sync_copy(ones_v, acc_sp.at[dstv.at[j]], add=True)

        plsc.subcore_barrier()
        pltpu.sync_copy(acc_sp.at[pl.ds(s * RPS, RPS)],
                        out_hbm.at[c].at[pl.ds(s * RPS, RPS)])

    return k(zeros16, dst3)


def _segsum_pass(table, src3, dst3, zeros, D):
    @functools.partial(
        pl.kernel,
        out_type=jax.ShapeDtypeStruct((NC, NP, D), jnp.float32),
        mesh=_vmesh(),
        scratch_types=[
            pltpu.VMEM((CH, CHUNK), jnp.int32),
            pltpu.VMEM((CH, CHUNK), jnp.int32),
            pltpu.VMEM((CHUNK, D), jnp.float32),
            pltpu.VMEM_SHARED((NP, D), jnp.float32),
            pltpu.SemaphoreType.DMA,
        ],
        compiler_params=_SC_PARAMS,
    )
    def k(table_hbm, src_hbm, dst_hbm, zeros_hbm, out_hbm,
          srcv, dstv, rows, acc_sp, sem):
        c = lax.axis_index("c")
        s = lax.axis_index("s")
        w = c * NS + s
        pltpu.sync_copy(zeros_hbm.at[pl.ds(s * RPS, RPS)],
                        acc_sp.at[pl.ds(s * RPS, RPS)])
        pltpu.sync_copy(src_hbm.at[w], srcv)
        pltpu.sync_copy(dst_hbm.at[w], dstv)
        plsc.subcore_barrier()

        @pl.loop(0, CH)
        def _(j):
            pltpu.async_copy(table_hbm.at[srcv.at[j]], rows, sem).wait()
            pltpu.sync_copy(rows, acc_sp.at[dstv.at[j]], add=True)

        plsc.subcore_barrier()
        pltpu.sync_copy(acc_sp.at[pl.ds(s * RPS, RPS)],
                        out_hbm.at[c].at[pl.ds(s * RPS, RPS)])

    return k(table, src3, dst3, zeros)


def _tc_matmul(x, W):
    K, M = W.shape

    def body(x_ref, w_ref, o_ref):
        o_ref[...] = jnp.dot(x_ref[...], w_ref[...],
                             preferred_element_type=jnp.float32)

    return pl.pallas_call(
        body,
        grid=(N // BLK,),
        in_specs=[pl.BlockSpec((BLK, K), lambda i: (i, 0)),
                  pl.BlockSpec((K, M), lambda i: (0, 0))],
        out_specs=pl.BlockSpec((BLK, M), lambda i: (i, 0)),
        out_shape=jax.ShapeDtypeStruct((N, M), jnp.float32),
    )(x, W)


def _tc_scale(deg_parts, y1):
    def body(deg_ref, y_ref, dinv_ref, ys_ref):
        d = deg_ref[0, :, 0] + deg_ref[1, :, 0] + 1.0
        dinv = lax.rsqrt(jnp.maximum(d, 1.0))
        dinv_ref[...] = dinv[:, None]
        ys_ref[...] = y_ref[...] * dinv[:, None]

    return pl.pallas_call(
        body,
        grid=(N // BLK,),
        in_specs=[pl.BlockSpec((NC, BLK, 16), lambda i: (0, i, 0)),
                  pl.BlockSpec((BLK, 64), lambda i: (i, 0))],
        out_specs=[pl.BlockSpec((BLK, 1), lambda i: (i, 0)),
                   pl.BlockSpec((BLK, 64), lambda i: (i, 0))],
        out_shape=[jax.ShapeDtypeStruct((N, 1), jnp.float32),
                   jax.ShapeDtypeStruct((N, 64), jnp.float32)],
    )(deg_parts, y1)


def _tc_layer2(acc1, y1s, dinv, b1, W2):
    def body(acc_ref, ys_ref, dinv_ref, b_ref, w_ref, o_ref):
        t = acc_ref[0] + acc_ref[1] + ys_ref[...]
        h1 = jnp.maximum(t * dinv_ref[...] + b_ref[...], 0.0)
        y2 = jnp.dot(h1, w_ref[...], preferred_element_type=jnp.float32)
        o_ref[...] = y2 * dinv_ref[...]

    return pl.pallas_call(
        body,
        grid=(N // BLK,),
        in_specs=[pl.BlockSpec((NC, BLK, 64), lambda i: (0, i, 0)),
                  pl.BlockSpec((BLK, 64), lambda i: (i, 0)),
                  pl.BlockSpec((BLK, 1), lambda i: (i, 0)),
                  pl.BlockSpec((1, 64), lambda i: (0, 0)),
                  pl.BlockSpec((64, 32), lambda i: (0, 0))],
        out_specs=pl.BlockSpec((BLK, 32), lambda i: (i, 0)),
        out_shape=jax.ShapeDtypeStruct((N, 32), jnp.float32),
    )(acc1, y1s, dinv, b1, W2)


def _tc_finalize(acc2, y2s, dinv, b2, wa_row, ba):
    def body(acc_ref, ys_ref, dinv_ref, b_ref, wa_ref, ba_ref, o_ref):
        t = acc_ref[0] + acc_ref[1] + ys_ref[...]
        h2 = jnp.maximum(t * dinv_ref[...] + b_ref[...], 0.0)
        logit = jnp.sum(h2 * wa_ref[...], axis=1, keepdims=True) + ba_ref[...]
        o_ref[...] = h2 * jax.nn.sigmoid(logit)

    return pl.pallas_call(
        body,
        grid=(N // BLK,),
        in_specs=[pl.BlockSpec((NC, BLK, 32), lambda i: (0, i, 0)),
                  pl.BlockSpec((BLK, 32), lambda i: (i, 0)),
                  pl.BlockSpec((BLK, 1), lambda i: (i, 0)),
                  pl.BlockSpec((1, 32), lambda i: (0, 0)),
                  pl.BlockSpec((1, 32), lambda i: (0, 0)),
                  pl.BlockSpec((1, 1), lambda i: (0, 0))],
        out_specs=pl.BlockSpec((BLK, 32), lambda i: (i, 0)),
        out_shape=jax.ShapeDtypeStruct((N, 32), jnp.float32),
    )(acc2, y2s, dinv, b2, wa_row, ba)


def kernel(x, edge_index, W1, b1, W2, b2, Wa, ba):
    src3 = edge_index[0].reshape(NW, CH, CHUNK)
    dst3 = edge_index[1].reshape(NW, CH, CHUNK)
    zeros64 = jnp.zeros((NP, 64), jnp.float32)

    deg_parts = _deg_pass(zeros64[:, :16], dst3)[:, :N]
    y1 = _tc_matmul(x, W1)
    dinv, y1s = _tc_scale(deg_parts, y1)
    acc1 = _segsum_pass(y1s, src3, dst3, zeros64, 64)[:, :N]
    y2s = _tc_layer2(acc1, y1s, dinv, b1.reshape(1, 64), W2)
    acc2 = _segsum_pass(y2s, src3, dst3, zeros64[:, :32], 32)[:, :N]
    return _tc_finalize(acc2, y2s, dinv, b2.reshape(1, 32),
                        Wa.reshape(1, 32), ba.reshape(1, 1))


# speedup vs baseline: 38.8629x; 1.6365x over previous
import functools

import jax
import jax.numpy as jnp
from jax import lax
from jax.experimental import pallas as pl
from jax.experimental.pallas import tpu as pltpu
from jax.experimental.pallas import tpu_sc as plsc

N = 10000
E = 320000
NC = 2
NS = 16
NW = NC * NS
CHUNK = 80
CH = E // NW // CHUNK
NP = 10240
RPS = NP // NS
NBUF = 5
BLK = 1000


def _vmesh():
    return plsc.VectorSubcoreMesh(core_axis_name="c", subcore_axis_name="s")


_SC_PARAMS = pltpu.CompilerParams(use_tc_tiling_on_sc=False)


def _deg_pass(zeros16, dst3):
    @functools.partial(
        pl.kernel,
        out_type=jax.ShapeDtypeStruct((NC, NP, 16), jnp.float32),
        mesh=_vmesh(),
        scratch_types=[
            pltpu.VMEM((CH, CHUNK), jnp.int32),
            pltpu.VMEM((CHUNK, 16), jnp.float32),
            pltpu.VMEM_SHARED((NP, 16), jnp.float32),
        ],
        compiler_params=_SC_PARAMS,
    )
    def k(zeros_hbm, dst_hbm, out_hbm, dstv, ones_v, acc_sp):
        c = lax.axis_index("c")
        s = lax.axis_index("s")
        w = c * NS + s
        row = jnp.where(lax.iota(jnp.int32, 16) == 0, 1.0, 0.0).astype(jnp.float32)

        @pl.loop(0, CHUNK)
        def _(i):
            ones_v[i] = row

        pltpu.sync_copy(zeros_hbm.at[pl.ds(s * RPS, RPS)],
                        acc_sp.at[pl.ds(s * RPS, RPS)])
        pltpu.sync_copy(dst_hbm.at[w], dstv)
        plsc.subcore_barrier()

        @pl.loop(0, CH)
        def _(j):
            pltpu.sync_copy(ones_v, acc_sp.at[dstv.at[j]], add=True)

        plsc.subcore_barrier()
        pltpu.sync_copy(acc_sp.at[pl.ds(s * RPS, RPS)],
                        out_hbm.at[c].at[pl.ds(s * RPS, RPS)])

    return k(zeros16, dst3)


def _segsum_pass(table, src3, dst3, zeros, D):
    @functools.partial(
        pl.kernel,
        out_type=jax.ShapeDtypeStruct((NC, NP, D), jnp.float32),
        mesh=_vmesh(),
        scratch_types=[
            pltpu.VMEM((CH, CHUNK), jnp.int32),
            pltpu.VMEM((CH, CHUNK), jnp.int32),
            pltpu.VMEM((NBUF, CHUNK, D), jnp.float32),
            pltpu.VMEM_SHARED((NP, D), jnp.float32),
            pltpu.SemaphoreType.DMA((NBUF,)),
            pltpu.SemaphoreType.DMA((NBUF,)),
        ],
        compiler_params=_SC_PARAMS,
    )
    def k(table_hbm, src_hbm, dst_hbm, zeros_hbm, out_hbm,
          srcv, dstv, rows, acc_sp, gsem, ssem):
        c = lax.axis_index("c")
        s = lax.axis_index("s")
        w = c * NS + s
        pltpu.sync_copy(zeros_hbm.at[pl.ds(s * RPS, RPS)],
                        acc_sp.at[pl.ds(s * RPS, RPS)])
        pltpu.sync_copy(src_hbm.at[w], srcv)
        pltpu.sync_copy(dst_hbm.at[w], dstv)
        plsc.subcore_barrier()

        for b in range(NBUF):
            pltpu.async_copy(table_hbm.at[srcv.at[b]], rows.at[b], gsem.at[b])

        @pl.loop(0, CH, step=NBUF)
        def _(j):
            descs = []
            for b in range(NBUF):
                pltpu.make_async_copy(table_hbm.at[srcv.at[j + b]],
                                      rows.at[b], gsem.at[b]).wait()
                descs.append(pltpu.async_copy(
                    rows.at[b], acc_sp.at[dstv.at[j + b]], ssem.at[b],
                    add=True))
            for b in range(NBUF):
                descs[b].wait()

                @pl.when(j + NBUF + b < CH)
                def _():
                    pltpu.async_copy(table_hbm.at[srcv.at[j + NBUF + b]],
                                     rows.at[b], gsem.at[b])

        plsc.subcore_barrier()
        pltpu.sync_copy(acc_sp.at[pl.ds(s * RPS, RPS)],
                        out_hbm.at[c].at[pl.ds(s * RPS, RPS)])

    return k(table, src3, dst3, zeros)


def _tc_matmul(x, W):
    K, M = W.shape

    def body(x_ref, w_ref, o_ref):
        o_ref[...] = jnp.dot(x_ref[...], w_ref[...],
                             preferred_element_type=jnp.float32)

    return pl.pallas_call(
        body,
        grid=(N // BLK,),
        in_specs=[pl.BlockSpec((BLK, K), lambda i: (i, 0)),
                  pl.BlockSpec((K, M), lambda i: (0, 0))],
        out_specs=pl.BlockSpec((BLK, M), lambda i: (i, 0)),
        out_shape=jax.ShapeDtypeStruct((N, M), jnp.float32),
    )(x, W)


def _tc_scale(deg_parts, y1):
    def body(deg_ref, y_ref, dinv_ref, ys_ref):
        d = deg_ref[0, :, 0] + deg_ref[1, :, 0] + 1.0
        dinv = lax.rsqrt(jnp.maximum(d, 1.0))
        dinv_ref[...] = dinv[:, None]
        ys_ref[...] = y_ref[...] * dinv[:, None]

    return pl.pallas_call(
        body,
        grid=(N // BLK,),
        in_specs=[pl.BlockSpec((NC, BLK, 16), lambda i: (0, i, 0)),
                  pl.BlockSpec((BLK, 64), lambda i: (i, 0))],
        out_specs=[pl.BlockSpec((BLK, 1), lambda i: (i, 0)),
                   pl.BlockSpec((BLK, 64), lambda i: (i, 0))],
        out_shape=[jax.ShapeDtypeStruct((N, 1), jnp.float32),
                   jax.ShapeDtypeStruct((N, 64), jnp.float32)],
    )(deg_parts, y1)


def _tc_layer2(acc1, y1s, dinv, b1, W2):
    def body(acc_ref, ys_ref, dinv_ref, b_ref, w_ref, o_ref):
        t = acc_ref[0] + acc_ref[1] + ys_ref[...]
        h1 = jnp.maximum(t * dinv_ref[...] + b_ref[...], 0.0)
        y2 = jnp.dot(h1, w_ref[...], preferred_element_type=jnp.float32)
        o_ref[...] = y2 * dinv_ref[...]

    return pl.pallas_call(
        body,
        grid=(N // BLK,),
        in_specs=[pl.BlockSpec((NC, BLK, 64), lambda i: (0, i, 0)),
                  pl.BlockSpec((BLK, 64), lambda i: (i, 0)),
                  pl.BlockSpec((BLK, 1), lambda i: (i, 0)),
                  pl.BlockSpec((1, 64), lambda i: (0, 0)),
                  pl.BlockSpec((64, 32), lambda i: (0, 0))],
        out_specs=pl.BlockSpec((BLK, 32), lambda i: (i, 0)),
        out_shape=jax.ShapeDtypeStruct((N, 32), jnp.float32),
    )(acc1, y1s, dinv, b1, W2)


def _tc_finalize(acc2, y2s, dinv, b2, wa_row, ba):
    def body(acc_ref, ys_ref, dinv_ref, b_ref, wa_ref, ba_ref, o_ref):
        t = acc_ref[0] + acc_ref[1] + ys_ref[...]
        h2 = jnp.maximum(t * dinv_ref[...] + b_ref[...], 0.0)
        logit = jnp.sum(h2 * wa_ref[...], axis=1, keepdims=True) + ba_ref[...]
        o_ref[...] = h2 * jax.nn.sigmoid(logit)

    return pl.pallas_call(
        body,
        grid=(N // BLK,),
        in_specs=[pl.BlockSpec((NC, BLK, 32), lambda i: (0, i, 0)),
                  pl.BlockSpec((BLK, 32), lambda i: (i, 0)),
                  pl.BlockSpec((BLK, 1), lambda i: (i, 0)),
                  pl.BlockSpec((1, 32), lambda i: (0, 0)),
                  pl.BlockSpec((1, 32), lambda i: (0, 0)),
                  pl.BlockSpec((1, 1), lambda i: (0, 0))],
        out_specs=pl.BlockSpec((BLK, 32), lambda i: (i, 0)),
        out_shape=jax.ShapeDtypeStruct((N, 32), jnp.float32),
    )(acc2, y2s, dinv, b2, wa_row, ba)


def kernel(x, edge_index, W1, b1, W2, b2, Wa, ba):
    src3 = edge_index[0].reshape(NW, CH, CHUNK)
    dst3 = edge_index[1].reshape(NW, CH, CHUNK)
    zeros64 = jnp.zeros((NP, 64), jnp.float32)

    deg_parts = _deg_pass(zeros64[:, :16], dst3)[:, :N]
    y1 = _tc_matmul(x, W1)
    dinv, y1s = _tc_scale(deg_parts, y1)
    acc1 = _segsum_pass(y1s, src3, dst3, zeros64, 64)[:, :N]
    y2s = _tc_layer2(acc1, y1s, dinv, b1.reshape(1, 64), W2)
    acc2 = _segsum_pass(y2s, src3, dst3, zeros64[:, :32], 32)[:, :N]
    return _tc_finalize(acc2, y2s, dinv, b2.reshape(1, 32),
                        Wa.reshape(1, 32), ba.reshape(1, 1))


# speedup vs baseline: 39.9445x; 1.0278x over previous
import functools

import jax
import jax.numpy as jnp
from jax import lax
from jax.experimental import pallas as pl
from jax.experimental.pallas import tpu as pltpu
from jax.experimental.pallas import tpu_sc as plsc

N = 10000
E = 320000
NC = 2
NS = 16
NW = NC * NS
CHUNK = 80
CH = E // NW // CHUNK
NP = 10240
RPS = NP // NS
NBUF = 5
BLK = 1000


def _vmesh():
    return plsc.VectorSubcoreMesh(core_axis_name="c", subcore_axis_name="s")


_SC_PARAMS = pltpu.CompilerParams(use_tc_tiling_on_sc=False)


def _deg_pass(zeros16, dst3):
    @functools.partial(
        pl.kernel,
        out_type=jax.ShapeDtypeStruct((NC, NP, 16), jnp.float32),
        mesh=_vmesh(),
        scratch_types=[
            pltpu.VMEM((CH, CHUNK), jnp.int32),
            pltpu.VMEM((CHUNK, 16), jnp.float32),
            pltpu.VMEM_SHARED((NP, 16), jnp.float32),
            pltpu.SemaphoreType.DMA((NBUF,)),
        ],
        compiler_params=_SC_PARAMS,
    )
    def k(zeros_hbm, dst_hbm, out_hbm, dstv, ones_v, acc_sp, ssem):
        c = lax.axis_index("c")
        s = lax.axis_index("s")
        w = c * NS + s
        row = jnp.where(lax.iota(jnp.int32, 16) == 0, 1.0, 0.0).astype(jnp.float32)

        @pl.loop(0, CHUNK)
        def _(i):
            ones_v[i] = row

        pltpu.sync_copy(zeros_hbm.at[pl.ds(s * RPS, RPS)],
                        acc_sp.at[pl.ds(s * RPS, RPS)])
        pltpu.sync_copy(dst_hbm.at[w], dstv)
        plsc.subcore_barrier()

        @pl.loop(0, CH, step=NBUF)
        def _(j):
            descs = [pltpu.async_copy(ones_v, acc_sp.at[dstv.at[j + b]],
                                      ssem.at[b], add=True)
                     for b in range(NBUF)]
            for d in descs:
                d.wait()

        plsc.subcore_barrier()
        pltpu.sync_copy(acc_sp.at[pl.ds(s * RPS, RPS)],
                        out_hbm.at[c].at[pl.ds(s * RPS, RPS)])

    return k(zeros16, dst3)


def _segsum_pass(table, src3, dst3, zeros, D):
    @functools.partial(
        pl.kernel,
        out_type=jax.ShapeDtypeStruct((NC, NP, D), jnp.float32),
        mesh=_vmesh(),
        scratch_types=[
            pltpu.VMEM((CH, CHUNK), jnp.int32),
            pltpu.VMEM((CH, CHUNK), jnp.int32),
            pltpu.VMEM((NBUF, CHUNK, D), jnp.float32),
            pltpu.VMEM_SHARED((NP, D), jnp.float32),
            pltpu.SemaphoreType.DMA((NBUF,)),
            pltpu.SemaphoreType.DMA((NBUF,)),
        ],
        compiler_params=_SC_PARAMS,
    )
    def k(table_hbm, src_hbm, dst_hbm, zeros_hbm, out_hbm,
          srcv, dstv, rows, acc_sp, gsem, ssem):
        c = lax.axis_index("c")
        s = lax.axis_index("s")
        w = c * NS + s
        pltpu.sync_copy(zeros_hbm.at[pl.ds(s * RPS, RPS)],
                        acc_sp.at[pl.ds(s * RPS, RPS)])
        pltpu.sync_copy(src_hbm.at[w], srcv)
        pltpu.sync_copy(dst_hbm.at[w], dstv)
        plsc.subcore_barrier()

        for b in range(NBUF):
            pltpu.async_copy(table_hbm.at[srcv.at[b]], rows.at[b], gsem.at[b])

        @pl.loop(0, CH, step=NBUF)
        def _(j):
            descs = []
            for b in range(NBUF):
                pltpu.make_async_copy(table_hbm.at[srcv.at[j + b]],
                                      rows.at[b], gsem.at[b]).wait()
                descs.append(pltpu.async_copy(
                    rows.at[b], acc_sp.at[dstv.at[j + b]], ssem.at[b],
                    add=True))
            for b in range(NBUF):
                descs[b].wait()

                @pl.when(j + NBUF + b < CH)
                def _():
                    pltpu.async_copy(table_hbm.at[srcv.at[j + NBUF + b]],
                                     rows.at[b], gsem.at[b])

        plsc.subcore_barrier()
        pltpu.sync_copy(acc_sp.at[pl.ds(s * RPS, RPS)],
                        out_hbm.at[c].at[pl.ds(s * RPS, RPS)])

    return k(table, src3, dst3, zeros)


def _tc_layer1(x, W1, deg_parts):
    def body(x_ref, w_ref, deg_ref, dinv_ref, ys_ref):
        d = deg_ref[0, :, 0] + deg_ref[1, :, 0] + 1.0
        dinv = lax.rsqrt(jnp.maximum(d, 1.0))
        dinv_ref[...] = dinv[:, None]
        y = jnp.dot(x_ref[...], w_ref[...], preferred_element_type=jnp.float32)
        ys_ref[...] = y * dinv[:, None]

    return pl.pallas_call(
        body,
        grid=(N // BLK,),
        in_specs=[pl.BlockSpec((BLK, 128), lambda i: (i, 0)),
                  pl.BlockSpec((128, 64), lambda i: (0, 0)),
                  pl.BlockSpec((NC, BLK, 16), lambda i: (0, i, 0))],
        out_specs=[pl.BlockSpec((BLK, 1), lambda i: (i, 0)),
                   pl.BlockSpec((BLK, 64), lambda i: (i, 0))],
        out_shape=[jax.ShapeDtypeStruct((N, 1), jnp.float32),
                   jax.ShapeDtypeStruct((N, 64), jnp.float32)],
    )(x, W1, deg_parts)


def _tc_layer2(acc1, y1s, dinv, b1, W2):
    def body(acc_ref, ys_ref, dinv_ref, b_ref, w_ref, o_ref):
        t = acc_ref[0] + acc_ref[1] + ys_ref[...]
        h1 = jnp.maximum(t * dinv_ref[...] + b_ref[...], 0.0)
        y2 = jnp.dot(h1, w_ref[...], preferred_element_type=jnp.float32)
        o_ref[...] = y2 * dinv_ref[...]

    return pl.pallas_call(
        body,
        grid=(N // BLK,),
        in_specs=[pl.BlockSpec((NC, BLK, 64), lambda i: (0, i, 0)),
                  pl.BlockSpec((BLK, 64), lambda i: (i, 0)),
                  pl.BlockSpec((BLK, 1), lambda i: (i, 0)),
                  pl.BlockSpec((1, 64), lambda i: (0, 0)),
                  pl.BlockSpec((64, 32), lambda i: (0, 0))],
        out_specs=pl.BlockSpec((BLK, 32), lambda i: (i, 0)),
        out_shape=jax.ShapeDtypeStruct((N, 32), jnp.float32),
    )(acc1, y1s, dinv, b1, W2)


def _tc_finalize(acc2, y2s, dinv, b2, wa_row, ba):
    def body(acc_ref, ys_ref, dinv_ref, b_ref, wa_ref, ba_ref, o_ref):
        t = acc_ref[0] + acc_ref[1] + ys_ref[...]
        h2 = jnp.maximum(t * dinv_ref[...] + b_ref[...], 0.0)
        logit = jnp.sum(h2 * wa_ref[...], axis=1, keepdims=True) + ba_ref[...]
        o_ref[...] = h2 * jax.nn.sigmoid(logit)

    return pl.pallas_call(
        body,
        grid=(N // BLK,),
        in_specs=[pl.BlockSpec((NC, BLK, 32), lambda i: (0, i, 0)),
                  pl.BlockSpec((BLK, 32), lambda i: (i, 0)),
                  pl.BlockSpec((BLK, 1), lambda i: (i, 0)),
                  pl.BlockSpec((1, 32), lambda i: (0, 0)),
                  pl.BlockSpec((1, 32), lambda i: (0, 0)),
                  pl.BlockSpec((1, 1), lambda i: (0, 0))],
        out_specs=pl.BlockSpec((BLK, 32), lambda i: (i, 0)),
        out_shape=jax.ShapeDtypeStruct((N, 32), jnp.float32),
    )(acc2, y2s, dinv, b2, wa_row, ba)


def kernel(x, edge_index, W1, b1, W2, b2, Wa, ba):
    src3 = edge_index[0].reshape(NW, CH, CHUNK)
    dst3 = edge_index[1].reshape(NW, CH, CHUNK)
    zeros64 = jnp.zeros((NP, 64), jnp.float32)

    deg_parts = _deg_pass(zeros64[:, :16], dst3)[:, :N]
    dinv, y1s = _tc_layer1(x, W1, deg_parts)
    acc1 = _segsum_pass(y1s, src3, dst3, zeros64, 64)[:, :N]
    y2s = _tc_layer2(acc1, y1s, dinv, b1.reshape(1, 64), W2)
    acc2 = _segsum_pass(y2s, src3, dst3, zeros64[:, :32], 32)[:, :N]
    return _tc_finalize(acc2, y2s, dinv, b2.reshape(1, 32),
                        Wa.reshape(1, 32), ba.reshape(1, 1))


# speedup vs baseline: 47.2589x; 1.1831x over previous
import functools

import numpy as np

import jax
import jax.numpy as jnp
from jax import lax
from jax.experimental import pallas as pl
from jax.experimental.pallas import tpu as pltpu
from jax.experimental.pallas import tpu_sc as plsc

N = 10000
E = 320000
NC = 2
NS = 16
NW = NC * NS
CHUNK = 80
CH = E // NW // CHUNK
NP = 10240
RPS = NP // NS
NBUF = 5
BLK = 2000

_Z16 = np.zeros((10240, 16), np.float32)
_Z32 = np.zeros((10240, 32), np.float32)
_Z64 = np.zeros((10240, 64), np.float32)


def _vmesh():
    return plsc.VectorSubcoreMesh(core_axis_name="c", subcore_axis_name="s")


_SC_PARAMS = pltpu.CompilerParams(use_tc_tiling_on_sc=False)


def _deg_pass(edges4):
    @functools.partial(
        pl.kernel,
        out_type=jax.ShapeDtypeStruct((NC, NP, 16), jnp.float32),
        mesh=_vmesh(),
        scratch_types=[
            pltpu.VMEM((CH, CHUNK), jnp.int32),
            pltpu.VMEM((CHUNK, 16), jnp.float32),
            pltpu.VMEM_SHARED((NP, 16), jnp.float32),
            pltpu.SemaphoreType.DMA((NBUF,)),
        ],
        compiler_params=_SC_PARAMS,
    )
    def k(zeros_hbm, er_hbm, out_hbm, dstv, ones_v, acc_sp, ssem):
        c = lax.axis_index("c")
        s = lax.axis_index("s")
        w = c * NS + s
        row = jnp.where(lax.iota(jnp.int32, 16) == 0, 1.0, 0.0).astype(jnp.float32)

        @pl.loop(0, CHUNK)
        def _(i):
            ones_v[i] = row

        pltpu.sync_copy(zeros_hbm.at[pl.ds(s * RPS, RPS)],
                        acc_sp.at[pl.ds(s * RPS, RPS)])
        pltpu.sync_copy(er_hbm.at[1].at[w], dstv)
        plsc.subcore_barrier()

        @pl.loop(0, CH, step=NBUF)
        def _(j):
            descs = [pltpu.async_copy(ones_v, acc_sp.at[dstv.at[j + b]],
                                      ssem.at[b], add=True)
                     for b in range(NBUF)]
            for d in descs:
                d.wait()

        plsc.subcore_barrier()
        pltpu.sync_copy(acc_sp.at[pl.ds(s * RPS, RPS)],
                        out_hbm.at[c].at[pl.ds(s * RPS, RPS)])

    return k(_Z16, edges4)


def _segsum_pass(table, edges4, D):
    @functools.partial(
        pl.kernel,
        out_type=jax.ShapeDtypeStruct((NC, NP, D), jnp.float32),
        mesh=_vmesh(),
        scratch_types=[
            pltpu.VMEM((CH, CHUNK), jnp.int32),
            pltpu.VMEM((CH, CHUNK), jnp.int32),
            pltpu.VMEM((NBUF, CHUNK, D), jnp.float32),
            pltpu.VMEM_SHARED((NP, D), jnp.float32),
            pltpu.SemaphoreType.DMA((NBUF,)),
            pltpu.SemaphoreType.DMA((NBUF,)),
        ],
        compiler_params=_SC_PARAMS,
    )
    def k(table_hbm, er_hbm, zeros_hbm, out_hbm,
          srcv, dstv, rows, acc_sp, gsem, ssem):
        c = lax.axis_index("c")
        s = lax.axis_index("s")
        w = c * NS + s
        pltpu.sync_copy(zeros_hbm.at[pl.ds(s * RPS, RPS)],
                        acc_sp.at[pl.ds(s * RPS, RPS)])
        pltpu.sync_copy(er_hbm.at[0].at[w], srcv)
        pltpu.sync_copy(er_hbm.at[1].at[w], dstv)
        plsc.subcore_barrier()

        for b in range(NBUF):
            pltpu.async_copy(table_hbm.at[srcv.at[b]], rows.at[b], gsem.at[b])

        @pl.loop(0, CH, step=NBUF)
        def _(j):
            descs = []
            for b in range(NBUF):
                pltpu.make_async_copy(table_hbm.at[srcv.at[j + b]],
                                      rows.at[b], gsem.at[b]).wait()
                descs.append(pltpu.async_copy(
                    rows.at[b], acc_sp.at[dstv.at[j + b]], ssem.at[b],
                    add=True))
            for b in range(NBUF):
                descs[b].wait()

                @pl.when(j + NBUF + b < CH)
                def _():
                    pltpu.async_copy(table_hbm.at[srcv.at[j + NBUF + b]],
                                     rows.at[b], gsem.at[b])

        plsc.subcore_barrier()
        pltpu.sync_copy(acc_sp.at[pl.ds(s * RPS, RPS)],
                        out_hbm.at[c].at[pl.ds(s * RPS, RPS)])

    return k(table, edges4, _Z64 if D == 64 else _Z32)


def _tc_layer1(x, W1, deg_parts):
    def body(x_ref, w_ref, deg_ref, dinv_ref, ys_ref):
        d = deg_ref[0, :, 0] + deg_ref[1, :, 0] + 1.0
        dinv = lax.rsqrt(jnp.maximum(d, 1.0))
        dinv_ref[...] = dinv[:, None]
        y = jnp.dot(x_ref[...], w_ref[...], preferred_element_type=jnp.float32)
        ys_ref[...] = y * dinv[:, None]

    return pl.pallas_call(
        body,
        grid=(N // BLK,),
        in_specs=[pl.BlockSpec((BLK, 128), lambda i: (i, 0)),
                  pl.BlockSpec((128, 64), lambda i: (0, 0)),
                  pl.BlockSpec((NC, BLK, 16), lambda i: (0, i, 0))],
        out_specs=[pl.BlockSpec((BLK, 1), lambda i: (i, 0)),
                   pl.BlockSpec((BLK, 64), lambda i: (i, 0))],
        out_shape=[jax.ShapeDtypeStruct((N, 1), jnp.float32),
                   jax.ShapeDtypeStruct((N, 64), jnp.float32)],
    )(x, W1, deg_parts)


def _tc_layer2(acc1, y1s, dinv, b1, W2):
    def body(acc_ref, ys_ref, dinv_ref, b_ref, w_ref, o_ref):
        t = acc_ref[0] + acc_ref[1] + ys_ref[...]
        h1 = jnp.maximum(t * dinv_ref[...] + b_ref[...], 0.0)
        y2 = jnp.dot(h1, w_ref[...], preferred_element_type=jnp.float32)
        o_ref[...] = y2 * dinv_ref[...]

    return pl.pallas_call(
        body,
        grid=(N // BLK,),
        in_specs=[pl.BlockSpec((NC, BLK, 64), lambda i: (0, i, 0)),
                  pl.BlockSpec((BLK, 64), lambda i: (i, 0)),
                  pl.BlockSpec((BLK, 1), lambda i: (i, 0)),
                  pl.BlockSpec((1, 64), lambda i: (0, 0)),
                  pl.BlockSpec((64, 32), lambda i: (0, 0))],
        out_specs=pl.BlockSpec((BLK, 32), lambda i: (i, 0)),
        out_shape=jax.ShapeDtypeStruct((N, 32), jnp.float32),
    )(acc1, y1s, dinv, b1, W2)


def _tc_finalize(acc2, y2s, dinv, b2, wa_row, ba):
    def body(acc_ref, ys_ref, dinv_ref, b_ref, wa_ref, ba_ref, o_ref):
        t = acc_ref[0] + acc_ref[1] + ys_ref[...]
        h2 = jnp.maximum(t * dinv_ref[...] + b_ref[...], 0.0)
        logit = jnp.sum(h2 * wa_ref[...], axis=1, keepdims=True) + ba_ref[...]
        o_ref[...] = h2 * jax.nn.sigmoid(logit)

    return pl.pallas_call(
        body,
        grid=(N // BLK,),
        in_specs=[pl.BlockSpec((NC, BLK, 32), lambda i: (0, i, 0)),
                  pl.BlockSpec((BLK, 32), lambda i: (i, 0)),
                  pl.BlockSpec((BLK, 1), lambda i: (i, 0)),
                  pl.BlockSpec((1, 32), lambda i: (0, 0)),
                  pl.BlockSpec((1, 32), lambda i: (0, 0)),
                  pl.BlockSpec((1, 1), lambda i: (0, 0))],
        out_specs=pl.BlockSpec((BLK, 32), lambda i: (i, 0)),
        out_shape=jax.ShapeDtypeStruct((N, 32), jnp.float32),
    )(acc2, y2s, dinv, b2, wa_row, ba)


def kernel(x, edge_index, W1, b1, W2, b2, Wa, ba):
    edges4 = edge_index.reshape(2, NW, CH, CHUNK)

    deg_parts = _deg_pass(edges4)
    dinv, y1s = _tc_layer1(x, W1, deg_parts)
    acc1 = _segsum_pass(y1s, edges4, 64)
    y2s = _tc_layer2(acc1, y1s, dinv, b1.reshape(1, 64), W2)
    acc2 = _segsum_pass(y2s, edges4, 32)
    return _tc_finalize(acc2, y2s, dinv, b2.reshape(1, 32),
                        Wa.reshape(1, 32), ba.reshape(1, 1))


# speedup vs baseline: 47.2629x; 1.0001x over previous
import functools

import numpy as np

import jax
import jax.numpy as jnp
from jax import lax
from jax.experimental import pallas as pl
from jax.experimental.pallas import tpu as pltpu
from jax.experimental.pallas import tpu_sc as plsc

N = 10000
E = 320000
NC = 2
NS = 16
NW = NC * NS
CHUNK = 80
CH = E // NW // CHUNK
NP = 10240
RPS = NP // NS
NBUF = 5
BLK = 2000

_Z16 = np.zeros((NP, 16), np.float32)
_Z32 = np.zeros((NP, 32), np.float32)
_Z64 = np.zeros((NP, 64), np.float32)


def _vmesh():
    return plsc.VectorSubcoreMesh(core_axis_name="c", subcore_axis_name="s")


_SC_PARAMS = pltpu.CompilerParams(use_tc_tiling_on_sc=False)


def _deg_pass(edges4):
    @functools.partial(
        pl.kernel,
        out_type=jax.ShapeDtypeStruct((NC, NP, 16), jnp.float32),
        mesh=_vmesh(),
        scratch_types=[
            pltpu.VMEM((CH, CHUNK), jnp.int32),
            pltpu.VMEM((CHUNK, 16), jnp.float32),
            pltpu.VMEM_SHARED((NP, 16), jnp.float32),
            pltpu.SemaphoreType.DMA((NBUF,)),
        ],
        compiler_params=_SC_PARAMS,
    )
    def k(zeros_hbm, er_hbm, out_hbm, dstv, ones_v, acc_sp, ssem):
        c = lax.axis_index("c")
        s = lax.axis_index("s")
        w = c * NS + s
        row = jnp.where(lax.iota(jnp.int32, 16) == 0, 1.0, 0.0).astype(jnp.float32)

        @pl.loop(0, CHUNK)
        def _(i):
            ones_v[i] = row

        pltpu.sync_copy(zeros_hbm.at[pl.ds(s * RPS, RPS)],
                        acc_sp.at[pl.ds(s * RPS, RPS)])
        pltpu.sync_copy(er_hbm.at[1].at[w], dstv)
        plsc.subcore_barrier()

        @pl.loop(0, CH, step=NBUF)
        def _(j):
            descs = [pltpu.async_copy(ones_v, acc_sp.at[dstv.at[j + b]],
                                      ssem.at[b], add=True)
                     for b in range(NBUF)]
            for d in descs:
                d.wait()

        plsc.subcore_barrier()
        pltpu.sync_copy(acc_sp.at[pl.ds(s * RPS, RPS)],
                        out_hbm.at[c].at[pl.ds(s * RPS, RPS)])

    return k(_Z16, edges4)


def _segsum_pass(table, edges4, D):
    @functools.partial(
        pl.kernel,
        out_type=jax.ShapeDtypeStruct((NC, NP, D), jnp.float32),
        mesh=_vmesh(),
        scratch_types=[
            pltpu.VMEM((CH, CHUNK), jnp.int32),
            pltpu.VMEM((CH, CHUNK), jnp.int32),
            pltpu.VMEM((NBUF, CHUNK, D), jnp.float32),
            pltpu.VMEM_SHARED((NP, D), jnp.float32),
            pltpu.SemaphoreType.DMA((NBUF,)),
            pltpu.SemaphoreType.DMA((NBUF,)),
        ],
        compiler_params=_SC_PARAMS,
    )
    def k(table_hbm, er_hbm, zeros_hbm, out_hbm,
          srcv, dstv, rows, acc_sp, gsem, ssem):
        c = lax.axis_index("c")
        s = lax.axis_index("s")
        w = c * NS + s
        pltpu.sync_copy(zeros_hbm.at[pl.ds(s * RPS, RPS)],
                        acc_sp.at[pl.ds(s * RPS, RPS)])
        pltpu.sync_copy(er_hbm.at[0].at[w], srcv)
        pltpu.sync_copy(er_hbm.at[1].at[w], dstv)
        plsc.subcore_barrier()

        for b in range(NBUF):
            pltpu.async_copy(table_hbm.at[srcv.at[b]], rows.at[b], gsem.at[b])

        @pl.loop(0, CH, step=NBUF)
        def _(j):
            descs = []
            for b in range(NBUF):
                pltpu.make_async_copy(table_hbm.at[srcv.at[j + b]],
                                      rows.at[b], gsem.at[b]).wait()
                descs.append(pltpu.async_copy(
                    rows.at[b], acc_sp.at[dstv.at[j + b]], ssem.at[b],
                    add=True))
            for b in range(NBUF):
                descs[b].wait()

                @pl.when(j + NBUF + b < CH)
                def _():
                    pltpu.async_copy(table_hbm.at[srcv.at[j + NBUF + b]],
                                     rows.at[b], gsem.at[b])

        plsc.subcore_barrier()
        pltpu.sync_copy(acc_sp.at[pl.ds(s * RPS, RPS)],
                        out_hbm.at[c].at[pl.ds(s * RPS, RPS)])

    return k(table, edges4, _Z64 if D == 64 else _Z32)


def _tc_layer1(x, W1, deg_parts):
    def body(x_ref, w_ref, deg_ref, dinv_ref, ys_ref):
        d = deg_ref[0, :, 0] + deg_ref[1, :, 0] + 1.0
        dinv = lax.rsqrt(jnp.maximum(d, 1.0))
        dinv_ref[...] = dinv[:, None]
        y = jnp.dot(x_ref[...], w_ref[...], preferred_element_type=jnp.float32)
        ys_ref[...] = y * dinv[:, None]

    return pl.pallas_call(
        body,
        grid=(N // BLK,),
        in_specs=[pl.BlockSpec((BLK, 128), lambda i: (i, 0)),
                  pl.BlockSpec((128, 64), lambda i: (0, 0)),
                  pl.BlockSpec((NC, BLK, 16), lambda i: (0, i, 0))],
        out_specs=[pl.BlockSpec((BLK, 1), lambda i: (i, 0)),
                   pl.BlockSpec((BLK, 64), lambda i: (i, 0))],
        out_shape=[jax.ShapeDtypeStruct((N, 1), jnp.float32),
                   jax.ShapeDtypeStruct((N, 64), jnp.float32)],
    )(x, W1, deg_parts)


def _tc_layer2(acc1, y1s, dinv, b1, W2):
    def body(acc_ref, ys_ref, dinv_ref, b_ref, w_ref, o_ref):
        t = acc_ref[0] + acc_ref[1] + ys_ref[...]
        h1 = jnp.maximum(t * dinv_ref[...] + b_ref[...], 0.0)
        y2 = jnp.dot(h1, w_ref[...], preferred_element_type=jnp.float32)
        o_ref[...] = y2 * dinv_ref[...]

    return pl.pallas_call(
        body,
        grid=(N // BLK,),
        in_specs=[pl.BlockSpec((NC, BLK, 64), lambda i: (0, i, 0)),
                  pl.BlockSpec((BLK, 64), lambda i: (i, 0)),
                  pl.BlockSpec((BLK, 1), lambda i: (i, 0)),
                  pl.BlockSpec((1, 64), lambda i: (0, 0)),
                  pl.BlockSpec((64, 32), lambda i: (0, 0))],
        out_specs=pl.BlockSpec((BLK, 32), lambda i: (i, 0)),
        out_shape=jax.ShapeDtypeStruct((N, 32), jnp.float32),
    )(acc1, y1s, dinv, b1, W2)


def _tc_finalize(acc2, y2s, dinv, b2, wa_row, ba):
    def body(acc_ref, ys_ref, dinv_ref, b_ref, wa_ref, ba_ref, o_ref):
        t = acc_ref[0] + acc_ref[1] + ys_ref[...]
        h2 = jnp.maximum(t * dinv_ref[...] + b_ref[...], 0.0)
        logit = jnp.sum(h2 * wa_ref[...], axis=1, keepdims=True) + ba_ref[...]
        o_ref[...] = h2 * jax.nn.sigmoid(logit)

    return pl.pallas_call(
        body,
        grid=(N // BLK,),
        in_specs=[pl.BlockSpec((NC, BLK, 32), lambda i: (0, i, 0)),
                  pl.BlockSpec((BLK, 32), lambda i: (i, 0)),
                  pl.BlockSpec((BLK, 1), lambda i: (i, 0)),
                  pl.BlockSpec((1, 32), lambda i: (0, 0)),
                  pl.BlockSpec((1, 32), lambda i: (0, 0)),
                  pl.BlockSpec((1, 1), lambda i: (0, 0))],
        out_specs=pl.BlockSpec((BLK, 32), lambda i: (i, 0)),
        out_shape=jax.ShapeDtypeStruct((N, 32), jnp.float32),
    )(acc2, y2s, dinv, b2, wa_row, ba)


def kernel(x, edge_index, W1, b1, W2, b2, Wa, ba):
    edges4 = edge_index.reshape(2, NW, CH, CHUNK)

    deg_parts = _deg_pass(edges4)
    dinv, y1s = _tc_layer1(x, W1, deg_parts)
    acc1 = _segsum_pass(y1s, edges4, 64)
    y2s = _tc_layer2(acc1, y1s, dinv, b1.reshape(1, 64), W2)
    acc2 = _segsum_pass(y2s, edges4, 32)
    return _tc_finalize(acc2, y2s, dinv, b2.reshape(1, 32),
                        Wa.reshape(1, 32), ba.reshape(1, 1))
